# baseline (speedup 1.0000x reference)
import functools

import jax
import jax.numpy as jnp
from jax import lax
from jax.experimental import pallas as pl
from jax.experimental.pallas import tpu as pltpu
from jax.experimental.pallas import tpu_sc as plsc

N_NODES = 10000
D_FEAT = 128
N_EDGES = 320000

NUM_CORES = 2
NUM_SUBCORES = 16
NUM_WORKERS = NUM_CORES * NUM_SUBCORES
EDGES_PER_WORKER = N_EDGES // NUM_WORKERS
CHUNK = 80
NUM_CHUNKS = EDGES_PER_WORKER // CHUNK
LANES = 16
GROUPS = CHUNK // LANES
STRIDE = LANES + 1
SETS = 5

_mesh = plsc.VectorSubcoreMesh(core_axis_name="c", subcore_axis_name="s")


@functools.partial(
    pl.kernel,
    out_type=jax.ShapeDtypeStruct((N_EDGES,), jnp.float32),
    mesh=_mesh,
    compiler_params=pltpu.CompilerParams(needs_layout_passes=False),
    scratch_types=[
        pltpu.VMEM((EDGES_PER_WORKER,), jnp.int32),
        pltpu.VMEM((EDGES_PER_WORKER,), jnp.int32),
        pltpu.VMEM((SETS, CHUNK, D_FEAT), jnp.float32),
        pltpu.VMEM((SETS, CHUNK, D_FEAT), jnp.float32),
        pltpu.VMEM((SETS, CHUNK), jnp.float32),
        pltpu.VMEM((GROUPS * LANES * STRIDE,), jnp.float32),
        pltpu.SemaphoreType.DMA((SETS,)),
        pltpu.SemaphoreType.DMA((SETS,)),
        pltpu.SemaphoreType.DMA((SETS,)),
    ],
)
def _decode(z_hbm, ei_hbm, out_hbm, idx_s, idx_d, src_v, dst_v, out_v,
            acc_v, sem_s, sem_d, sem_o):
    w = lax.axis_index("s") * NUM_CORES + lax.axis_index("c")
    w_base = pl.multiple_of(w * EDGES_PER_WORKER, 8)

    pltpu.sync_copy(ei_hbm.at[pl.ds(w_base, EDGES_PER_WORKER)], idx_s)
    pltpu.sync_copy(ei_hbm.at[pl.ds(N_EDGES + w_base, EDGES_PER_WORKER)],
                    idx_d)

    def start_gather(i, b):
        off = pl.multiple_of(i * CHUNK, 8)
        pltpu.async_copy(z_hbm.at[idx_s.at[pl.ds(off, CHUNK)]],
                         src_v.at[b], sem_s.at[b])
        pltpu.async_copy(z_hbm.at[idx_d.at[pl.ds(off, CHUNK)]],
                         dst_v.at[b], sem_d.at[b])

    def wait_gather(b):
        pltpu.make_async_copy(z_hbm.at[pl.ds(0, CHUNK)], src_v.at[b],
                              sem_s.at[b]).wait()
        pltpu.make_async_copy(z_hbm.at[pl.ds(0, CHUNK)], dst_v.at[b],
                              sem_d.at[b]).wait()

    def wait_out(b):
        pltpu.make_async_copy(out_v.at[b], out_hbm.at[pl.ds(0, CHUNK)],
                              sem_o.at[b]).wait()

    def start_out(i, b):
        base = pl.multiple_of(w_base + i * CHUNK, 8)
        pltpu.async_copy(out_v.at[b], out_hbm.at[pl.ds(base, CHUNK)],
                         sem_o.at[b])

    def compute(b):
        colbase = lax.iota(jnp.int32, LANES) * STRIDE

        def group_body(g):
            abase = g * (LANES * STRIDE)
            for j in range(LANES):
                e = g * LANES + j
                pa = (src_v[b, e, pl.ds(0, LANES)]
                      * dst_v[b, e, pl.ds(0, LANES)])
                pb = (src_v[b, e, pl.ds(4 * LANES, LANES)]
                      * dst_v[b, e, pl.ds(4 * LANES, LANES)])
                for k in range(1, 4):
                    pa += (src_v[b, e, pl.ds(k * LANES, LANES)]
                           * dst_v[b, e, pl.ds(k * LANES, LANES)])
                    pb += (src_v[b, e, pl.ds((k + 4) * LANES, LANES)]
                           * dst_v[b, e, pl.ds((k + 4) * LANES, LANES)])
                acc_v[pl.ds(abase + j * STRIDE, LANES)] = pa + pb
            colidx = colbase + abase
            tot = plsc.load_gather(acc_v, [colidx])
            for k in range(1, LANES):
                tot += plsc.load_gather(acc_v, [colidx + k])
            out_v[b, pl.ds(g * LANES, LANES)] = 1.0 / (1.0 + jnp.exp(-tot))

        for g in range(GROUPS):
            group_body(g)

    for b in range(SETS):
        start_gather(b, b)

    def loop_body(i, carry):
        b = lax.rem(i, SETS)
        wait_gather(b)

        @pl.when(i >= SETS)
        def _():
            wait_out(b)

        compute(b)
        start_out(i, b)

        @pl.when(i + SETS < NUM_CHUNKS)
        def _():
            start_gather(i + SETS, b)

        return carry

    lax.fori_loop(0, NUM_CHUNKS, loop_body, 0)

    for b in range(SETS):
        wait_out(b)


def kernel(z, edge_index):
    return _decode(z, edge_index.astype(jnp.int32).reshape(-1))


# speedup vs baseline: 1.4731x; 1.4731x over previous
import functools

import jax
import jax.numpy as jnp
from jax import lax
from jax.experimental import pallas as pl
from jax.experimental.pallas import tpu as pltpu
from jax.experimental.pallas import tpu_sc as plsc

N_NODES = 10000
D_FEAT = 128
N_EDGES = 320000

NUM_CORES = 2
NUM_SUBCORES = 16
NUM_WORKERS = NUM_CORES * NUM_SUBCORES
EDGES_PER_WORKER = N_EDGES // NUM_WORKERS
CHUNK = 80
NUM_CHUNKS = EDGES_PER_WORKER // CHUNK
LANES = 16
GROUPS = CHUNK // LANES
STRIDE = LANES + 1
SETS = 5

_mesh = plsc.VectorSubcoreMesh(core_axis_name="c", subcore_axis_name="s")


@functools.partial(
    pl.kernel,
    out_type=jax.ShapeDtypeStruct((N_EDGES,), jnp.float32),
    mesh=_mesh,
    compiler_params=pltpu.CompilerParams(needs_layout_passes=False),
    scratch_types=[
        pltpu.VMEM((EDGES_PER_WORKER,), jnp.int32),
        pltpu.VMEM((EDGES_PER_WORKER,), jnp.int32),
        pltpu.VMEM((SETS, CHUNK, D_FEAT), jnp.float32),
        pltpu.VMEM((SETS, CHUNK, D_FEAT), jnp.float32),
        pltpu.VMEM((SETS, CHUNK), jnp.float32),
        pltpu.VMEM((GROUPS * LANES * STRIDE,), jnp.float32),
        pltpu.SemaphoreType.DMA((SETS,)),
        pltpu.SemaphoreType.DMA((SETS,)),
        pltpu.SemaphoreType.DMA((SETS,)),
    ],
)
def _decode(z_hbm, ei_hbm, out_hbm, idx_s, idx_d, src_v, dst_v, out_v,
            acc_v, sem_s, sem_d, sem_o):
    w = lax.axis_index("s") * NUM_CORES + lax.axis_index("c")
    w_base = pl.multiple_of(w * EDGES_PER_WORKER, 8)

    pltpu.sync_copy(ei_hbm.at[pl.ds(w_base, EDGES_PER_WORKER)], idx_s)
    pltpu.sync_copy(ei_hbm.at[pl.ds(N_EDGES + w_base, EDGES_PER_WORKER)],
                    idx_d)

    def start_gather(i, b):
        off = pl.multiple_of(i * CHUNK, 8)
        pltpu.async_copy(z_hbm.at[idx_s.at[pl.ds(off, CHUNK)]],
                         src_v.at[b], sem_s.at[b])
        pltpu.async_copy(z_hbm.at[idx_d.at[pl.ds(off, CHUNK)]],
                         dst_v.at[b], sem_d.at[b])

    def wait_gather(b):
        pltpu.make_async_copy(z_hbm.at[pl.ds(0, CHUNK)], src_v.at[b],
                              sem_s.at[b]).wait()
        pltpu.make_async_copy(z_hbm.at[pl.ds(0, CHUNK)], dst_v.at[b],
                              sem_d.at[b]).wait()

    def wait_out(b):
        pltpu.make_async_copy(out_v.at[b], out_hbm.at[pl.ds(0, CHUNK)],
                              sem_o.at[b]).wait()

    def start_out(i, b):
        base = pl.multiple_of(w_base + i * CHUNK, 8)
        pltpu.async_copy(out_v.at[b], out_hbm.at[pl.ds(base, CHUNK)],
                         sem_o.at[b])

    def compute(b):
        colbase = lax.iota(jnp.int32, LANES) * STRIDE

        @plsc.parallel_loop(0, LANES, unroll=LANES)
        def lane_body(j):
            for g in range(GROUPS):
                e = g * LANES + j
                pa = (src_v[b, e, pl.ds(0, LANES)]
                      * dst_v[b, e, pl.ds(0, LANES)])
                pb = (src_v[b, e, pl.ds(4 * LANES, LANES)]
                      * dst_v[b, e, pl.ds(4 * LANES, LANES)])
                for k in range(1, 4):
                    pa += (src_v[b, e, pl.ds(k * LANES, LANES)]
                           * dst_v[b, e, pl.ds(k * LANES, LANES)])
                    pb += (src_v[b, e, pl.ds((k + 4) * LANES, LANES)]
                           * dst_v[b, e, pl.ds((k + 4) * LANES, LANES)])
                acc_v[pl.ds(g * (LANES * STRIDE) + j * STRIDE, LANES)] = pa + pb

        @plsc.parallel_loop(0, GROUPS, unroll=GROUPS)
        def group_body(g):
            colidx = colbase + g * (LANES * STRIDE)
            tot = plsc.load_gather(acc_v, [colidx])
            for k in range(1, LANES):
                tot += plsc.load_gather(acc_v, [colidx + k])
            out_v[b, pl.ds(g * LANES, LANES)] = 1.0 / (1.0 + jnp.exp(-tot))

    for b in range(SETS):
        start_gather(b, b)

    def loop_body(i, carry):
        b = lax.rem(i, SETS)
        wait_gather(b)

        @pl.when(i >= SETS)
        def _():
            wait_out(b)

        compute(b)
        start_out(i, b)

        @pl.when(i + SETS < NUM_CHUNKS)
        def _():
            start_gather(i + SETS, b)

        return carry

    lax.fori_loop(0, NUM_CHUNKS, loop_body, 0)

    for b in range(SETS):
        wait_out(b)


def kernel(z, edge_index):
    return _decode(z, edge_index.astype(jnp.int32).reshape(-1))
